# baseline (speedup 1.0000x reference)
import functools

import jax
import jax.numpy as jnp
from jax import lax
from jax.experimental import pallas as pl
from jax.experimental.pallas import tpu as pltpu
from jax.experimental.pallas import tpu_sc as plsc

N = 4096
D = 128
K = 8
R = 256
NB = N // R
CH = 16
NCH = N // CH
NF = N // 128
LN = 16
NW = 32
RPW = N // NW
NG = RPW // LN


def _tc_body(e_blk_ref, e_all_ref, d2f_ref, cmt_ref, rs_ref, sb_ref):
    i = pl.program_id(0)

    @pl.when(i == 0)
    def _():
        ea = e_all_ref[...]
        sq = ea * ea
        ones = jnp.ones((1, D), dtype=jnp.float32)
        sb_ref[...] = lax.dot_general(
            ones, sq, (((1,), (1,)), ((), ())),
            preferred_element_type=jnp.float32)
        colsum = jnp.sum(ea, axis=0, keepdims=True)
        colsum2 = jnp.sum(sq, axis=0, keepdims=True)
        var = (colsum2 - colsum * colsum * (1.0 / N)) * (1.0 / (N - 1))
        rs_ref[0, 0] = jnp.sum(var) * (1.0 / D)

    e_blk = e_blk_ref[...]
    sa = jnp.sum(e_blk * e_blk, axis=1, keepdims=True)
    g = lax.dot_general(
        e_blk, e_all_ref[...], (((1,), (1,)), ((), ())),
        preferred_element_type=jnp.float32)
    d2 = jnp.maximum(sa + sb_ref[...] - 2.0 * g, 0.0)
    for k in range(NF):
        d2f_ref[:, k, :] = d2[:, k * 128:(k + 1) * 128]
    m = jnp.min(d2.reshape(R // CH, CH, N), axis=1)
    for k in range(NF):
        cmt_ref[:, k, :] = m[:, k * 128:(k + 1) * 128]


def _bcast_i32(x):
    return jnp.zeros((LN,), jnp.int32) + x


def _argmin_pass(ref, m, col_idx, inf16):
    S = 4
    U = 2
    seg = m // S

    def body(j, carry):
        bests, bidxs = carry
        bests = list(bests)
        bidxs = list(bidxs)
        for u in range(U):
            for k in range(S):
                idx = j * U + u + k * seg
                v = plsc.load_gather(ref, [_bcast_i32(idx), col_idx])
                pred = v < bests[k]
                bests[k] = jnp.where(pred, v, bests[k])
                bidxs[k] = jnp.where(pred, _bcast_i32(idx), bidxs[k])
        return tuple(bests), tuple(bidxs)

    zero_i = jnp.zeros((LN,), jnp.int32)
    init = (tuple(inf16 for _ in range(S)), tuple(zero_i for _ in range(S)))
    bests, bidxs = lax.fori_loop(0, seg // U, body, init)
    best, bidx = bests[0], bidxs[0]
    for k in range(1, S):
        pred = bests[k] < best
        best = jnp.where(pred, bests[k], best)
        bidx = jnp.where(pred, bidxs[k], bidx)
    return best, bidx


def _sc_body(cmt_hbm, d2l_hbm, out_hbm, cm_ref, idx_ref, cand_ref, c2_ref,
             accv_ref, sem):
    c = lax.axis_index("c")
    s = lax.axis_index("s")
    wid = s * 2 + c
    lane = lax.broadcasted_iota(jnp.int32, (LN,), 0)
    inf16 = jnp.full((LN,), jnp.inf, jnp.float32)

    pltpu.sync_copy(cmt_hbm.at[:, wid], cm_ref)

    def group(lg, acc):
        col = lg * LN + lane
        row = wid * RPW + lg * LN + lane
        offs = []
        for q in range(K):
            best, bidx = _argmin_pass(cm_ref, NCH, col, inf16)
            plsc.store_scatter(cm_ref, [bidx, col], inf16)
            idx_ref[pl.ds(q * LN, LN)] = row * NF + bidx // 8
            offs.append((bidx % 8) * CH)
        pltpu.async_copy(d2l_hbm.at[idx_ref], cand_ref, sem).wait()
        for q in range(K):
            row_i = _bcast_i32(q * LN) + lane
            for t in range(CH):
                c2_ref[q * CH + t] = plsc.load_gather(
                    cand_ref, [row_i, offs[q] + t])
        for q in range(K):
            best, bidx = _argmin_pass(c2_ref, K * CH, lane, inf16)
            acc = acc + best
            plsc.store_scatter(c2_ref, [bidx, lane], inf16)
        return acc

    acc = lax.fori_loop(0, NG, group, jnp.zeros((LN,), jnp.float32))
    accv_ref[...] = acc
    pltpu.sync_copy(accv_ref, out_hbm.at[pl.ds(wid * LN, LN)])


@functools.partial(
    pl.kernel,
    out_type=jax.ShapeDtypeStruct((NW * LN,), jnp.float32),
    mesh=plsc.VectorSubcoreMesh(core_axis_name="c", subcore_axis_name="s"),
    compiler_params=pltpu.CompilerParams(
        use_tc_tiling_on_sc=False, needs_layout_passes=False),
    scratch_types=[
        pltpu.VMEM((NCH, 128), jnp.float32),
        pltpu.VMEM((K * LN,), jnp.int32),
        pltpu.VMEM((K * LN, 128), jnp.float32),
        pltpu.VMEM((K * CH, LN), jnp.float32),
        pltpu.VMEM((LN,), jnp.float32),
        pltpu.SemaphoreType.DMA,
    ],
)
def _sc_select(cmt_hbm, d2l_hbm, out_hbm, cm_ref, idx_ref, cand_ref, c2_ref,
               accv_ref, sem):
    _sc_body(cmt_hbm, d2l_hbm, out_hbm, cm_ref, idx_ref, cand_ref, c2_ref,
             accv_ref, sem)


def kernel(e, lp):
    del lp
    d2f, cmt, rs = pl.pallas_call(
        _tc_body,
        grid=(NB,),
        in_specs=[
            pl.BlockSpec((R, D), lambda i: (i, 0)),
            pl.BlockSpec((N, D), lambda i: (0, 0)),
        ],
        out_specs=[
            pl.BlockSpec((R, NF, 128), lambda i: (i, 0, 0)),
            pl.BlockSpec((R // CH, NF, 128), lambda i: (i, 0, 0)),
            pl.BlockSpec(memory_space=pltpu.SMEM),
        ],
        out_shape=[
            jax.ShapeDtypeStruct((N, NF, 128), jnp.float32),
            jax.ShapeDtypeStruct((NCH, NF, 128), jnp.float32),
            jax.ShapeDtypeStruct((1, 1), jnp.float32),
        ],
        scratch_shapes=[
            pltpu.VMEM((1, N), jnp.float32),
        ],
    )(e, e)
    d2l = d2f.reshape(N * NF, 128)
    partial = _sc_select(cmt, d2l)
    return (jnp.sum(partial) * (1.0 / (N * K))) / rs[0, 0]


# speedup vs baseline: 1.3161x; 1.3161x over previous
import functools

import jax
import jax.numpy as jnp
from jax import lax
from jax.experimental import pallas as pl
from jax.experimental.pallas import tpu as pltpu
from jax.experimental.pallas import tpu_sc as plsc

N = 4096
D = 128
K = 8
R = 256
NB = N // R
CH = 16
NCH = N // CH
NF = N // 128
LN = 16
NW = 32
RPW = N // NW
NG = RPW // LN


def _tc_body(e_blk_ref, e_all_ref, d2f_ref, cmt_ref, rs_ref, sb_ref):
    i = pl.program_id(0)

    @pl.when(i == 0)
    def _():
        ea = e_all_ref[...]
        sq = ea * ea
        ones = jnp.ones((1, D), dtype=jnp.float32)
        sb_ref[...] = lax.dot_general(
            ones, sq, (((1,), (1,)), ((), ())),
            preferred_element_type=jnp.float32)
        colsum = jnp.sum(ea, axis=0, keepdims=True)
        colsum2 = jnp.sum(sq, axis=0, keepdims=True)
        var = (colsum2 - colsum * colsum * (1.0 / N)) * (1.0 / (N - 1))
        rs_ref[0, 0] = jnp.sum(var) * (1.0 / D)

    e_blk = e_blk_ref[...]
    sa = jnp.sum(e_blk * e_blk, axis=1, keepdims=True)
    g = lax.dot_general(
        e_blk, e_all_ref[...], (((1,), (1,)), ((), ())),
        preferred_element_type=jnp.float32)
    d2 = jnp.maximum(sa + sb_ref[...] - 2.0 * g, 0.0)
    for k in range(NF):
        d2f_ref[:, k, :, :] = d2[:, k * 128:(k + 1) * 128].reshape(R // 8, 8, 128)
    m = jnp.min(d2.reshape(R // CH, CH, N), axis=1)
    for k in range(NF):
        cmt_ref[:, k, :, :] = m[:, k * 128:(k + 1) * 128].reshape(R // CH // 8, 8, 128)


def _bcast_i32(x):
    return jnp.zeros((LN,), jnp.int32) + x


def _argmin_pass(ref, m, col_idx, inf16):
    S = 4
    U = 2
    seg = m // S
    rank3 = len(ref.shape) == 3

    def load(idx):
        if rank3:
            return plsc.load_gather(
                ref, [_bcast_i32(idx >> 3), _bcast_i32(idx & 7), col_idx])
        return plsc.load_gather(ref, [_bcast_i32(idx), col_idx])

    def body(j, carry):
        bests, bidxs = carry
        bests = list(bests)
        bidxs = list(bidxs)
        for u in range(U):
            for k in range(S):
                idx = j * U + u + k * seg
                v = load(idx)
                pred = v < bests[k]
                bests[k] = jnp.where(pred, v, bests[k])
                bidxs[k] = jnp.where(pred, _bcast_i32(idx), bidxs[k])
        return tuple(bests), tuple(bidxs)

    zero_i = jnp.zeros((LN,), jnp.int32)
    init = (tuple(inf16 for _ in range(S)), tuple(zero_i for _ in range(S)))
    bests, bidxs = lax.fori_loop(0, seg // U, body, init)
    best, bidx = bests[0], bidxs[0]
    for k in range(1, S):
        pred = bests[k] < best
        best = jnp.where(pred, bests[k], best)
        bidx = jnp.where(pred, bidxs[k], bidx)
    return best, bidx


def _sc_body(cmt_hbm, d2l_hbm, out_hbm, cm_ref, idx_ref, cand_ref, c2_ref,
             accv_ref, sem):
    c = lax.axis_index("c")
    s = lax.axis_index("s")
    wid = s * 2 + c
    lane = lax.broadcasted_iota(jnp.int32, (LN,), 0)
    inf16 = jnp.full((LN,), jnp.inf, jnp.float32)

    pltpu.sync_copy(cmt_hbm.at[:, wid], cm_ref)

    def group(lg, acc):
        col = lg * LN + lane
        row = wid * RPW + lg * LN + lane
        offs = []
        for q in range(K):
            best, bidx = _argmin_pass(cm_ref, NCH, col, inf16)
            plsc.store_scatter(cm_ref, [bidx >> 3, bidx & 7, col], inf16)
            idx_ref[pl.ds(q * LN, LN)] = (
                (row >> 3) * (NF * 8) + (bidx >> 3) * 8 + (row & 7))
            offs.append((bidx & 7) * CH)
        pltpu.async_copy(d2l_hbm.at[idx_ref], cand_ref, sem).wait()
        for q in range(K):
            row_i = _bcast_i32(q * LN) + lane
            for t in range(CH):
                c2_ref[q * CH + t] = plsc.load_gather(
                    cand_ref, [row_i, offs[q] + t])
        for q in range(K):
            best, bidx = _argmin_pass(c2_ref, K * CH, lane, inf16)
            acc = acc + best
            plsc.store_scatter(c2_ref, [bidx, lane], inf16)
        return acc

    acc = lax.fori_loop(0, NG, group, jnp.zeros((LN,), jnp.float32))
    accv_ref[...] = acc
    pltpu.sync_copy(accv_ref, out_hbm.at[pl.ds(wid * LN, LN)])


@functools.partial(
    pl.kernel,
    out_type=jax.ShapeDtypeStruct((NW * LN,), jnp.float32),
    mesh=plsc.VectorSubcoreMesh(core_axis_name="c", subcore_axis_name="s"),
    compiler_params=pltpu.CompilerParams(
        use_tc_tiling_on_sc=False, needs_layout_passes=False),
    scratch_types=[
        pltpu.VMEM((NCH // 8, 8, 128), jnp.float32),
        pltpu.VMEM((K * LN,), jnp.int32),
        pltpu.VMEM((K * LN, 128), jnp.float32),
        pltpu.VMEM((K * CH, LN), jnp.float32),
        pltpu.VMEM((LN,), jnp.float32),
        pltpu.SemaphoreType.DMA,
    ],
)
def _sc_select(cmt_hbm, d2l_hbm, out_hbm, cm_ref, idx_ref, cand_ref, c2_ref,
               accv_ref, sem):
    _sc_body(cmt_hbm, d2l_hbm, out_hbm, cm_ref, idx_ref, cand_ref, c2_ref,
             accv_ref, sem)


def kernel(e, lp):
    del lp
    d2f, cmt, rs = pl.pallas_call(
        _tc_body,
        grid=(NB,),
        in_specs=[
            pl.BlockSpec((R, D), lambda i: (i, 0)),
            pl.BlockSpec((N, D), lambda i: (0, 0)),
        ],
        out_specs=[
            pl.BlockSpec((R // 8, NF, 8, 128), lambda i: (i, 0, 0, 0)),
            pl.BlockSpec((R // CH // 8, NF, 8, 128), lambda i: (i, 0, 0, 0)),
            pl.BlockSpec(memory_space=pltpu.SMEM),
        ],
        out_shape=[
            jax.ShapeDtypeStruct((N // 8, NF, 8, 128), jnp.float32),
            jax.ShapeDtypeStruct((NCH // 8, NF, 8, 128), jnp.float32),
            jax.ShapeDtypeStruct((1, 1), jnp.float32),
        ],
        scratch_shapes=[
            pltpu.VMEM((1, N), jnp.float32),
        ],
    )(e, e)
    d2l = d2f.reshape(N // 8 * NF * 8, 128)
    partial = _sc_select(cmt, d2l)
    return (jnp.sum(partial) * (1.0 / (N * K))) / rs[0, 0]


# speedup vs baseline: 1.7989x; 1.3669x over previous
import functools

import jax
import jax.numpy as jnp
from jax import lax
from jax.experimental import pallas as pl
from jax.experimental.pallas import tpu as pltpu
from jax.experimental.pallas import tpu_sc as plsc

N = 4096
D = 128
K = 8
R = 256
NB = N // R
CH = 16
NCH = N // CH
NF = N // 128
LN = 16
NW = 32
RPW = N // NW
NG = RPW // LN


def _tc_body(e_blk_ref, e_all_ref, d2f_ref, cmt_ref, rs_ref, sb_ref):
    i = pl.program_id(0)

    @pl.when(i == 0)
    def _():
        ea = e_all_ref[...]
        sq = ea * ea
        ones = jnp.ones((1, D), dtype=jnp.float32)
        sb_ref[...] = lax.dot_general(
            ones, sq, (((1,), (1,)), ((), ())),
            preferred_element_type=jnp.float32)
        colsum = jnp.sum(ea, axis=0, keepdims=True)
        colsum2 = jnp.sum(sq, axis=0, keepdims=True)
        var = (colsum2 - colsum * colsum * (1.0 / N)) * (1.0 / (N - 1))
        rs_ref[0, 0] = jnp.sum(var) * (1.0 / D)

    e_blk = e_blk_ref[...]
    sa = jnp.sum(e_blk * e_blk, axis=1, keepdims=True)
    g = lax.dot_general(
        e_blk, e_all_ref[...], (((1,), (1,)), ((), ())),
        preferred_element_type=jnp.float32)
    d2 = jnp.maximum(sa + sb_ref[...] - 2.0 * g, 0.0)
    for k in range(NF):
        d2f_ref[:, k, :, :] = d2[:, k * 128:(k + 1) * 128].reshape(R // 8, 8, 128)
    m = jnp.min(d2.reshape(R // CH, CH, N), axis=1)
    for k in range(NF):
        cmt_ref[:, k, :, :] = m[:, k * 128:(k + 1) * 128].reshape(R // CH // 8, 8, 128)


def _bcast_i32(x):
    return jnp.zeros((LN,), jnp.int32) + x


def _argmin_vecs(vs, inf16):
    S = 4 if len(vs) >= 8 else 1
    zero_i = jnp.zeros((LN,), jnp.int32)
    parts = []
    for k in range(S):
        best, bidx = None, None
        for pos in range(k, len(vs), S):
            if best is None:
                best, bidx = vs[pos], _bcast_i32(pos)
            else:
                pred = vs[pos] < best
                best = jnp.where(pred, vs[pos], best)
                bidx = jnp.where(pred, _bcast_i32(pos), bidx)
        parts.append((best, bidx))
    best, bidx = parts[0]
    for b2, i2 in parts[1:]:
        pred = b2 < best
        best = jnp.where(pred, b2, best)
        bidx = jnp.where(pred, i2, bidx)
    return best, bidx


def _min_tree(vs):
    while len(vs) > 1:
        vs = [jnp.minimum(a, b) for a, b in zip(vs[::2], vs[1::2])] + (
            [vs[-1]] if len(vs) % 2 else [])
    return vs[0]


def _sc_body(cmt_hbm, d2l_hbm, out_hbm, cm_ref, idx_ref, cand_ref, sm_ref,
             accv_ref, sem):
    c = lax.axis_index("c")
    s = lax.axis_index("s")
    wid = s * 2 + c
    lane = lax.broadcasted_iota(jnp.int32, (LN,), 0)
    inf16 = jnp.full((LN,), jnp.inf, jnp.float32)

    pltpu.sync_copy(cmt_hbm.at[:, wid], cm_ref)

    def group(lg, acc):
        col = lg * LN + lane
        row = wid * RPW + lg * LN + lane

        def build(j2, _):
            vs = [plsc.load_gather(cm_ref, [_bcast_i32(j2), _bcast_i32(t), col])
                  for t in range(8)]
            plsc.store_scatter(sm_ref, [_bcast_i32(j2), col], _min_tree(vs))
            return 0

        lax.fori_loop(0, NF, build, 0)

        offs = []
        mm = []
        for q in range(K):
            svals = [plsc.load_gather(sm_ref, [_bcast_i32(j2), col])
                     for j2 in range(NF)]
            _, sj = _argmin_vecs(svals, inf16)
            vs = [plsc.load_gather(cm_ref, [sj, _bcast_i32(t), col])
                  for t in range(8)]
            cbest, tq = _argmin_vecs(vs, inf16)
            plsc.store_scatter(cm_ref, [sj, tq, col], inf16)
            nm = _min_tree([jnp.where(tq == t, inf16, vs[t])
                            for t in range(8)])
            plsc.store_scatter(sm_ref, [sj, col], nm)
            idx_ref[pl.ds(q * LN, LN)] = (
                (row >> 3) * (NF * 8) + sj * 8 + (row & 7))
            offs.append(tq * CH)
            mm.append(cbest)
        pltpu.async_copy(d2l_hbm.at[idx_ref], cand_ref, sem).wait()
        for p in range(K):
            _, qb = _argmin_vecs(mm, inf16)
            rowi = qb * LN + lane
            offsel = offs[0]
            for q in range(1, K):
                offsel = jnp.where(qb == q, offs[q], offsel)
            vs = [plsc.load_gather(cand_ref, [rowi, offsel + t])
                  for t in range(CH)]
            vbest, ti = _argmin_vecs(vs, inf16)
            acc = acc + vbest
            plsc.store_scatter(cand_ref, [rowi, offsel + ti], inf16)
            nm = _min_tree([jnp.where(ti == t, inf16, vs[t])
                            for t in range(CH)])
            for q in range(K):
                mm[q] = jnp.where(qb == q, nm, mm[q])
        return acc

    acc = lax.fori_loop(0, NG, group, jnp.zeros((LN,), jnp.float32))
    accv_ref[...] = acc
    pltpu.sync_copy(accv_ref, out_hbm.at[pl.ds(wid * LN, LN)])


@functools.partial(
    pl.kernel,
    out_type=jax.ShapeDtypeStruct((NW * LN,), jnp.float32),
    mesh=plsc.VectorSubcoreMesh(core_axis_name="c", subcore_axis_name="s"),
    compiler_params=pltpu.CompilerParams(
        use_tc_tiling_on_sc=False, needs_layout_passes=False),
    scratch_types=[
        pltpu.VMEM((NCH // 8, 8, 128), jnp.float32),
        pltpu.VMEM((K * LN,), jnp.int32),
        pltpu.VMEM((K * LN, 128), jnp.float32),
        pltpu.VMEM((NF, 128), jnp.float32),
        pltpu.VMEM((LN,), jnp.float32),
        pltpu.SemaphoreType.DMA,
    ],
)
def _sc_select(cmt_hbm, d2l_hbm, out_hbm, cm_ref, idx_ref, cand_ref, sm_ref,
               accv_ref, sem):
    _sc_body(cmt_hbm, d2l_hbm, out_hbm, cm_ref, idx_ref, cand_ref, sm_ref,
             accv_ref, sem)


def kernel(e, lp):
    del lp
    d2f, cmt, rs = pl.pallas_call(
        _tc_body,
        grid=(NB,),
        in_specs=[
            pl.BlockSpec((R, D), lambda i: (i, 0)),
            pl.BlockSpec((N, D), lambda i: (0, 0)),
        ],
        out_specs=[
            pl.BlockSpec((R // 8, NF, 8, 128), lambda i: (i, 0, 0, 0)),
            pl.BlockSpec((R // CH // 8, NF, 8, 128), lambda i: (i, 0, 0, 0)),
            pl.BlockSpec(memory_space=pltpu.SMEM),
        ],
        out_shape=[
            jax.ShapeDtypeStruct((N // 8, NF, 8, 128), jnp.float32),
            jax.ShapeDtypeStruct((NCH // 8, NF, 8, 128), jnp.float32),
            jax.ShapeDtypeStruct((1, 1), jnp.float32),
        ],
        scratch_shapes=[
            pltpu.VMEM((1, N), jnp.float32),
        ],
    )(e, e)
    d2l = d2f.reshape(N // 8 * NF * 8, 128)
    partial = _sc_select(cmt, d2l)
    return (jnp.sum(partial) * (1.0 / (N * K))) / rs[0, 0]


# speedup vs baseline: 2.0009x; 1.1123x over previous
import functools

import jax
import jax.numpy as jnp
from jax import lax
from jax.experimental import pallas as pl
from jax.experimental.pallas import tpu as pltpu
from jax.experimental.pallas import tpu_sc as plsc

N = 4096
D = 128
K = 8
R = 256
NB = N // R
CH = 16
NCH = N // CH
NF = N // 128
LN = 16
NW = 32
RPW = N // NW
NG = RPW // LN
GL = K * LN


def _tc_body(e_blk_ref, e_all_ref, d2f_ref, cmt_ref, smt_ref, rs_ref, sb_ref):
    i = pl.program_id(0)

    @pl.when(i == 0)
    def _():
        ea = e_all_ref[...]
        sq = ea * ea
        ones = jnp.ones((1, D), dtype=jnp.float32)
        sb_ref[...] = lax.dot_general(
            ones, sq, (((1,), (1,)), ((), ())),
            preferred_element_type=jnp.float32)
        colsum = jnp.sum(ea, axis=0, keepdims=True)
        colsum2 = jnp.sum(sq, axis=0, keepdims=True)
        var = (colsum2 - colsum * colsum * (1.0 / N)) * (1.0 / (N - 1))
        rs_ref[0, 0] = jnp.sum(var) * (1.0 / D)

    e_blk = e_blk_ref[...]
    sa = jnp.sum(e_blk * e_blk, axis=1, keepdims=True)
    g = lax.dot_general(
        e_blk, e_all_ref[...], (((1,), (1,)), ((), ())),
        preferred_element_type=jnp.float32)
    d2 = jnp.maximum(sa + sb_ref[...] - 2.0 * g, 0.0)
    for k in range(NF):
        d2f_ref[:, k, :, :] = d2[:, k * 128:(k + 1) * 128].reshape(R // 8, 8, 128)
    m = jnp.min(d2.reshape(R // CH, CH, N), axis=1)
    for k in range(NF):
        cmt_ref[:, k, :, :] = m[:, k * 128:(k + 1) * 128].reshape(R // CH // 8, 8, 128)
    sm = jnp.min(m.reshape(R // 128, 8, N), axis=1)
    for k in range(NF):
        smt_ref[:, k, :] = sm[:, k * 128:(k + 1) * 128]


def _bcast_i32(x):
    return jnp.zeros((LN,), jnp.int32) + x


def _argmin_vecs(vs, inf16):
    S = 4 if len(vs) >= 8 else 1
    parts = []
    for k in range(S):
        best, bidx = None, None
        for pos in range(k, len(vs), S):
            if best is None:
                best, bidx = vs[pos], _bcast_i32(pos)
            else:
                pred = vs[pos] < best
                best = jnp.where(pred, vs[pos], best)
                bidx = jnp.where(pred, _bcast_i32(pos), bidx)
        parts.append((best, bidx))
    best, bidx = parts[0]
    for b2, i2 in parts[1:]:
        pred = b2 < best
        best = jnp.where(pred, b2, best)
        bidx = jnp.where(pred, i2, bidx)
    return best, bidx


def _min_tree(vs):
    while len(vs) > 1:
        vs = [jnp.minimum(a, b) for a, b in zip(vs[::2], vs[1::2])] + (
            [vs[-1]] if len(vs) % 2 else [])
    return vs[0]


def _sc_body(cmt_hbm, smt_hbm, d2l_hbm, out_hbm, cm_ref, sm_ref, idx_ref,
             cand_ref, accv_ref, sem):
    c = lax.axis_index("c")
    s = lax.axis_index("s")
    wid = s * 2 + c
    lane = lax.broadcasted_iota(jnp.int32, (LN,), 0)
    inf16 = jnp.full((LN,), jnp.inf, jnp.float32)

    pltpu.sync_copy(cmt_hbm.at[:, wid], cm_ref)
    pltpu.sync_copy(smt_hbm.at[:, wid], sm_ref)

    def p2fire(g):
        par = g & 1
        col = g * LN + lane
        row = wid * RPW + g * LN + lane
        offs = []
        mm = []
        for q in range(K):
            svals = [plsc.load_gather(sm_ref, [_bcast_i32(j2), col])
                     for j2 in range(NF)]
            _, sj = _argmin_vecs(svals, inf16)
            vs = [plsc.load_gather(cm_ref, [sj, _bcast_i32(t), col])
                  for t in range(8)]
            cbest, tq = _argmin_vecs(vs, inf16)
            plsc.store_scatter(cm_ref, [sj, tq, col], inf16)
            nm = _min_tree([jnp.where(tq == t, inf16, vs[t])
                            for t in range(8)])
            plsc.store_scatter(sm_ref, [sj, col], nm)
            idx_ref[pl.ds(par * GL + q * LN, LN)] = (
                (row >> 3) * (NF * 8) + sj * 8 + (row & 7))
            offs.append(tq * CH)
            mm.append(cbest)
        pltpu.async_copy(d2l_hbm.at[idx_ref.at[pl.ds(par * GL, GL)]],
                         cand_ref.at[pl.ds(par * GL, GL)], sem)
        return tuple(offs), tuple(mm)

    def wait_gather(g):
        par = g & 1
        pltpu.make_async_copy(d2l_hbm.at[idx_ref.at[pl.ds(par * GL, GL)]],
                              cand_ref.at[pl.ds(par * GL, GL)], sem).wait()

    def p3(g, offs, mm, acc):
        par = g & 1
        mm = list(mm)
        for p in range(K):
            _, qb = _argmin_vecs(mm, inf16)
            rowi = _bcast_i32(par * GL) + qb * LN + lane
            offsel = offs[0]
            for q in range(1, K):
                offsel = jnp.where(qb == q, offs[q], offsel)
            vs = [plsc.load_gather(cand_ref, [rowi, offsel + t])
                  for t in range(CH)]
            vbest, ti = _argmin_vecs(vs, inf16)
            acc = acc + vbest
            plsc.store_scatter(cand_ref, [rowi, offsel + ti], inf16)
            nm = _min_tree([jnp.where(ti == t, inf16, vs[t])
                            for t in range(CH)])
            for q in range(K):
                mm[q] = jnp.where(qb == q, nm, mm[q])
        return acc

    offs0, mm0 = p2fire(0)

    def body(i, carry):
        offs, mm, acc = carry
        offs2, mm2 = p2fire(i + 1)
        wait_gather(i)
        acc = p3(i, offs, mm, acc)
        return offs2, mm2, acc

    offs, mm, acc = lax.fori_loop(
        0, NG - 1, body, (offs0, mm0, jnp.zeros((LN,), jnp.float32)))
    wait_gather(NG - 1)
    acc = p3(NG - 1, offs, mm, acc)
    accv_ref[...] = acc
    pltpu.sync_copy(accv_ref, out_hbm.at[pl.ds(wid * LN, LN)])


@functools.partial(
    pl.kernel,
    out_type=jax.ShapeDtypeStruct((NW * LN,), jnp.float32),
    mesh=plsc.VectorSubcoreMesh(core_axis_name="c", subcore_axis_name="s"),
    compiler_params=pltpu.CompilerParams(
        use_tc_tiling_on_sc=False, needs_layout_passes=False),
    scratch_types=[
        pltpu.VMEM((NCH // 8, 8, 128), jnp.float32),
        pltpu.VMEM((NF, 128), jnp.float32),
        pltpu.VMEM((2 * GL,), jnp.int32),
        pltpu.VMEM((2 * GL, 128), jnp.float32),
        pltpu.VMEM((LN,), jnp.float32),
        pltpu.SemaphoreType.DMA,
    ],
)
def _sc_select(cmt_hbm, smt_hbm, d2l_hbm, out_hbm, cm_ref, sm_ref, idx_ref,
               cand_ref, accv_ref, sem):
    _sc_body(cmt_hbm, smt_hbm, d2l_hbm, out_hbm, cm_ref, sm_ref, idx_ref,
             cand_ref, accv_ref, sem)


def kernel(e, lp):
    del lp
    d2f, cmt, smt, rs = pl.pallas_call(
        _tc_body,
        grid=(NB,),
        in_specs=[
            pl.BlockSpec((R, D), lambda i: (i, 0)),
            pl.BlockSpec((N, D), lambda i: (0, 0)),
        ],
        out_specs=[
            pl.BlockSpec((R // 8, NF, 8, 128), lambda i: (i, 0, 0, 0)),
            pl.BlockSpec((R // CH // 8, NF, 8, 128), lambda i: (i, 0, 0, 0)),
            pl.BlockSpec((R // 128, NF, 128), lambda i: (i, 0, 0)),
            pl.BlockSpec(memory_space=pltpu.SMEM),
        ],
        out_shape=[
            jax.ShapeDtypeStruct((N // 8, NF, 8, 128), jnp.float32),
            jax.ShapeDtypeStruct((NCH // 8, NF, 8, 128), jnp.float32),
            jax.ShapeDtypeStruct((N // 128, NF, 128), jnp.float32),
            jax.ShapeDtypeStruct((1, 1), jnp.float32),
        ],
        scratch_shapes=[
            pltpu.VMEM((1, N), jnp.float32),
        ],
    )(e, e)
    d2l = d2f.reshape(N // 8 * NF * 8, 128)
    partial = _sc_select(cmt, smt, d2l)
    return (jnp.sum(partial) * (1.0 / (N * K))) / rs[0, 0]


# speedup vs baseline: 2.1155x; 1.0572x over previous
import functools

import jax
import jax.numpy as jnp
from jax import lax
from jax.experimental import pallas as pl
from jax.experimental.pallas import tpu as pltpu
from jax.experimental.pallas import tpu_sc as plsc

N = 4096
D = 128
K = 8
R = 256
NB = N // R
CH = 16
NCH = N // CH
NF = N // 128
LN = 16
NW = 32
RPW = N // NW
NG = RPW // LN
GL = K * LN


def _tc_body(e_blk_ref, e_all_ref, d2f_ref, cmt_ref, smt_ref, rs_ref, sb_ref):
    i = pl.program_id(0)

    @pl.when(i == 0)
    def _():
        ea = e_all_ref[...].astype(jnp.float32)
        sq = ea * ea
        ones = jnp.ones((1, D), dtype=jnp.float32)
        sb_ref[...] = lax.dot_general(
            ones, sq, (((1,), (1,)), ((), ())),
            preferred_element_type=jnp.float32)
        colsum = jnp.sum(ea, axis=0, keepdims=True)
        colsum2 = jnp.sum(sq, axis=0, keepdims=True)
        var = (colsum2 - colsum * colsum * (1.0 / N)) * (1.0 / (N - 1))
        rs_ref[0, 0] = jnp.sum(var) * (1.0 / D)

    e_blk = e_blk_ref[...].astype(jnp.float32)
    sa = jnp.sum(e_blk * e_blk, axis=1, keepdims=True)
    g = lax.dot_general(
        e_blk_ref[...], e_all_ref[...], (((1,), (1,)), ((), ())),
        preferred_element_type=jnp.float32)
    d2 = jnp.maximum(sa + sb_ref[...] - 2.0 * g, 0.0)
    for k in range(NF):
        d2f_ref[:, k, :, :] = d2[:, k * 128:(k + 1) * 128].reshape(R // 8, 8, 128)
    m = jnp.min(d2.reshape(R // 128, CH, 8, N), axis=1).reshape(R // CH, N)
    for k in range(NF):
        cmt_ref[:, k, :, :] = m[:, k * 128:(k + 1) * 128].reshape(R // CH // 8, 8, 128)
    sm = jnp.min(m.reshape(R // 128, 8, N), axis=1)
    for k in range(NF):
        smt_ref[:, k, :] = sm[:, k * 128:(k + 1) * 128]


def _bcast_i32(x):
    return jnp.zeros((LN,), jnp.int32) + x


def _argmin_vecs(vs, inf16):
    S = 4 if len(vs) >= 8 else 1
    parts = []
    for k in range(S):
        best, bidx = None, None
        for pos in range(k, len(vs), S):
            if best is None:
                best, bidx = vs[pos], _bcast_i32(pos)
            else:
                pred = vs[pos] < best
                best = jnp.where(pred, vs[pos], best)
                bidx = jnp.where(pred, _bcast_i32(pos), bidx)
        parts.append((best, bidx))
    best, bidx = parts[0]
    for b2, i2 in parts[1:]:
        pred = b2 < best
        best = jnp.where(pred, b2, best)
        bidx = jnp.where(pred, i2, bidx)
    return best, bidx


def _min_tree(vs):
    while len(vs) > 1:
        vs = [jnp.minimum(a, b) for a, b in zip(vs[::2], vs[1::2])] + (
            [vs[-1]] if len(vs) % 2 else [])
    return vs[0]


def _sc_body(cmt_hbm, smt_hbm, d2l_hbm, out_hbm, cm_ref, sm_ref, idx_ref,
             cand_ref, accv_ref, sem):
    c = lax.axis_index("c")
    s = lax.axis_index("s")
    wid = s * 2 + c
    lane = lax.broadcasted_iota(jnp.int32, (LN,), 0)
    inf16 = jnp.full((LN,), jnp.inf, jnp.float32)

    pltpu.sync_copy(cmt_hbm.at[:, wid], cm_ref)
    pltpu.sync_copy(smt_hbm.at[:, wid], sm_ref)

    def p2fire(g):
        par = g & 1
        col = g * LN + lane
        row = wid * RPW + g * LN + lane
        offs = []
        mm = []
        for q in range(K):
            svals = [plsc.load_gather(sm_ref, [_bcast_i32(j2), col])
                     for j2 in range(NF)]
            _, sj = _argmin_vecs(svals, inf16)
            vs = [plsc.load_gather(cm_ref, [sj, _bcast_i32(t), col])
                  for t in range(8)]
            cbest, tq = _argmin_vecs(vs, inf16)
            plsc.store_scatter(cm_ref, [sj, tq, col], inf16)
            nm = _min_tree([jnp.where(tq == t, inf16, vs[t])
                            for t in range(8)])
            plsc.store_scatter(sm_ref, [sj, col], nm)
            idx_ref[pl.ds(par * GL + q * LN, LN)] = (
                (row >> 3) * (NF * 8) + sj * 8 + (row & 7))
            offs.append(tq)
            mm.append(cbest)
        pltpu.async_copy(d2l_hbm.at[idx_ref.at[pl.ds(par * GL, GL)]],
                         cand_ref.at[pl.ds(par * GL, GL)], sem)
        return tuple(offs), tuple(mm)

    def wait_gather(g):
        par = g & 1
        pltpu.make_async_copy(d2l_hbm.at[idx_ref.at[pl.ds(par * GL, GL)]],
                              cand_ref.at[pl.ds(par * GL, GL)], sem).wait()

    def p3(g, offs, mm, acc):
        par = g & 1
        mm = list(mm)
        for p in range(K):
            _, qb = _argmin_vecs(mm, inf16)
            rowi = _bcast_i32(par * GL) + qb * LN + lane
            offsel = offs[0]
            for q in range(1, K):
                offsel = jnp.where(qb == q, offs[q], offsel)
            vs = [plsc.load_gather(cand_ref, [rowi, offsel + t * 8])
                  for t in range(CH)]
            vbest, ti = _argmin_vecs(vs, inf16)
            acc = acc + vbest
            plsc.store_scatter(cand_ref, [rowi, offsel + ti * 8], inf16)
            nm = _min_tree([jnp.where(ti == t, inf16, vs[t])
                            for t in range(CH)])
            for q in range(K):
                mm[q] = jnp.where(qb == q, nm, mm[q])
        return acc

    offs0, mm0 = p2fire(0)

    def body(i, carry):
        offs, mm, acc = carry
        offs2, mm2 = p2fire(i + 1)
        wait_gather(i)
        acc = p3(i, offs, mm, acc)
        return offs2, mm2, acc

    offs, mm, acc = lax.fori_loop(
        0, NG - 1, body, (offs0, mm0, jnp.zeros((LN,), jnp.float32)))
    wait_gather(NG - 1)
    acc = p3(NG - 1, offs, mm, acc)
    accv_ref[...] = acc
    pltpu.sync_copy(accv_ref, out_hbm.at[pl.ds(wid * LN, LN)])


@functools.partial(
    pl.kernel,
    out_type=jax.ShapeDtypeStruct((NW * LN,), jnp.float32),
    mesh=plsc.VectorSubcoreMesh(core_axis_name="c", subcore_axis_name="s"),
    compiler_params=pltpu.CompilerParams(
        use_tc_tiling_on_sc=False, needs_layout_passes=False),
    scratch_types=[
        pltpu.VMEM((NCH // 8, 8, 128), jnp.float32),
        pltpu.VMEM((NF, 128), jnp.float32),
        pltpu.VMEM((2 * GL,), jnp.int32),
        pltpu.VMEM((2 * GL, 128), jnp.float32),
        pltpu.VMEM((LN,), jnp.float32),
        pltpu.SemaphoreType.DMA,
    ],
)
def _sc_select(cmt_hbm, smt_hbm, d2l_hbm, out_hbm, cm_ref, sm_ref, idx_ref,
               cand_ref, accv_ref, sem):
    _sc_body(cmt_hbm, smt_hbm, d2l_hbm, out_hbm, cm_ref, sm_ref, idx_ref,
             cand_ref, accv_ref, sem)


def kernel(e, lp):
    del lp
    e_bf = e.astype(jnp.bfloat16)
    d2f, cmt, smt, rs = pl.pallas_call(
        _tc_body,
        grid=(NB,),
        in_specs=[
            pl.BlockSpec((R, D), lambda i: (i, 0)),
            pl.BlockSpec((N, D), lambda i: (0, 0)),
        ],
        out_specs=[
            pl.BlockSpec((R // 8, NF, 8, 128), lambda i: (i, 0, 0, 0)),
            pl.BlockSpec((R // CH // 8, NF, 8, 128), lambda i: (i, 0, 0, 0)),
            pl.BlockSpec((R // 128, NF, 128), lambda i: (i, 0, 0)),
            pl.BlockSpec(memory_space=pltpu.SMEM),
        ],
        out_shape=[
            jax.ShapeDtypeStruct((N // 8, NF, 8, 128), jnp.float32),
            jax.ShapeDtypeStruct((NCH // 8, NF, 8, 128), jnp.float32),
            jax.ShapeDtypeStruct((N // 128, NF, 128), jnp.float32),
            jax.ShapeDtypeStruct((1, 1), jnp.float32),
        ],
        scratch_shapes=[
            pltpu.VMEM((1, N), jnp.float32),
        ],
    )(e_bf, e_bf)
    d2l = d2f.reshape(N // 8 * NF * 8, 128)
    partial = _sc_select(cmt, smt, d2l)
    return (jnp.sum(partial) * (1.0 / (N * K))) / rs[0, 0]


# speedup vs baseline: 2.1545x; 1.0184x over previous
import functools

import jax
import jax.numpy as jnp
from jax import lax
from jax.experimental import pallas as pl
from jax.experimental.pallas import tpu as pltpu
from jax.experimental.pallas import tpu_sc as plsc

N = 4096
D = 128
K = 8
R = 256
NB = N // R
CH = 16
NCH = N // CH
NF = N // 128
LN = 16
NW = 32
RPW = N // NW
NG = RPW // LN
GL = K * LN


def _tc_body(e_blk_ref, e_all_ref, d2f_ref, cmt_ref, smt_ref, rs_ref, sb_ref):
    i = pl.program_id(0)

    @pl.when(i == 0)
    def _():
        ea = e_all_ref[...].astype(jnp.float32)
        sq = ea * ea
        ones = jnp.ones((1, D), dtype=jnp.float32)
        sb_ref[...] = lax.dot_general(
            ones, sq, (((1,), (1,)), ((), ())),
            preferred_element_type=jnp.float32)
        colsum = jnp.sum(ea, axis=0, keepdims=True)
        colsum2 = jnp.sum(sq, axis=0, keepdims=True)
        var = (colsum2 - colsum * colsum * (1.0 / N)) * (1.0 / (N - 1))
        rs_ref[0, 0] = jnp.sum(var) * (1.0 / D)

    e_blk = e_blk_ref[...].astype(jnp.float32)
    sa = jnp.sum(e_blk * e_blk, axis=1, keepdims=True)
    g = lax.dot_general(
        e_blk_ref[...], e_all_ref[...], (((1,), (1,)), ((), ())),
        preferred_element_type=jnp.float32)
    d2 = sa + sb_ref[...] - 2.0 * g
    for k in range(NF):
        d2f_ref[:, k, :, :] = d2[:, k * 128:(k + 1) * 128].reshape(R // 8, 8, 128)
    m = jnp.min(d2.reshape(R // 128, CH, 8, N), axis=1).reshape(R // CH, N)
    for k in range(NF):
        cmt_ref[:, k, :, :] = m[:, k * 128:(k + 1) * 128].reshape(R // CH // 8, 8, 128)
    sm = jnp.min(m.reshape(R // 128, 8, N), axis=1)
    for k in range(NF):
        smt_ref[:, k, :] = sm[:, k * 128:(k + 1) * 128]


def _bcast_i32(x):
    return jnp.zeros((LN,), jnp.int32) + x


def _argmin_vecs(vs, inf16):
    S = 4 if len(vs) >= 8 else 1
    parts = []
    for k in range(S):
        best, bidx = None, None
        for pos in range(k, len(vs), S):
            if best is None:
                best, bidx = vs[pos], _bcast_i32(pos)
            else:
                pred = vs[pos] < best
                best = jnp.where(pred, vs[pos], best)
                bidx = jnp.where(pred, _bcast_i32(pos), bidx)
        parts.append((best, bidx))
    best, bidx = parts[0]
    for b2, i2 in parts[1:]:
        pred = b2 < best
        best = jnp.where(pred, b2, best)
        bidx = jnp.where(pred, i2, bidx)
    return best, bidx


def _min_tree(vs):
    while len(vs) > 1:
        vs = [jnp.minimum(a, b) for a, b in zip(vs[::2], vs[1::2])] + (
            [vs[-1]] if len(vs) % 2 else [])
    return vs[0]


def _sc_body(cmt_hbm, smt_hbm, d2l_hbm, out_hbm, cm_ref, sm_ref, idx_ref,
             cand_ref, accv_ref, sem):
    c = lax.axis_index("c")
    s = lax.axis_index("s")
    wid = s * 2 + c
    lane = lax.broadcasted_iota(jnp.int32, (LN,), 0)
    inf16 = jnp.full((LN,), jnp.inf, jnp.float32)

    pltpu.sync_copy(cmt_hbm.at[:, wid], cm_ref)
    pltpu.sync_copy(smt_hbm.at[:, wid], sm_ref)

    def p2fire(g):
        par = g & 1
        col = g * LN + lane
        row = wid * RPW + g * LN + lane
        offs = []
        mm = []
        for q in range(K):
            svals = [plsc.load_gather(sm_ref, [_bcast_i32(j2), col])
                     for j2 in range(NF)]
            _, sj = _argmin_vecs(svals, inf16)
            vs = [plsc.load_gather(cm_ref, [sj, _bcast_i32(t), col])
                  for t in range(8)]
            cbest, tq = _argmin_vecs(vs, inf16)
            plsc.store_scatter(cm_ref, [sj, tq, col], inf16)
            nm = _min_tree([jnp.where(tq == t, inf16, vs[t])
                            for t in range(8)])
            plsc.store_scatter(sm_ref, [sj, col], nm)
            idx_ref[pl.ds(par * GL + q * LN, LN)] = (
                (row >> 3) * (NF * 8) + sj * 8 + (row & 7))
            offs.append(tq)
            mm.append(cbest)
        pltpu.async_copy(d2l_hbm.at[idx_ref.at[pl.ds(par * GL, GL)]],
                         cand_ref.at[pl.ds(par * GL, GL)], sem)
        return tuple(offs), tuple(mm)

    def wait_gather(g):
        par = g & 1
        pltpu.make_async_copy(d2l_hbm.at[idx_ref.at[pl.ds(par * GL, GL)]],
                              cand_ref.at[pl.ds(par * GL, GL)], sem).wait()

    def p3(g, offs, mm, acc):
        par = g & 1
        mm = list(mm)
        for p in range(K):
            _, qb = _argmin_vecs(mm, inf16)
            rowi = _bcast_i32(par * GL) + qb * LN + lane
            offsel = offs[0]
            for q in range(1, K):
                offsel = jnp.where(qb == q, offs[q], offsel)
            vs = [plsc.load_gather(cand_ref, [rowi, offsel + t * 8])
                  for t in range(CH)]
            vbest, ti = _argmin_vecs(vs, inf16)
            acc = acc + vbest
            plsc.store_scatter(cand_ref, [rowi, offsel + ti * 8], inf16)
            nm = _min_tree([jnp.where(ti == t, inf16, vs[t])
                            for t in range(CH)])
            for q in range(K):
                mm[q] = jnp.where(qb == q, nm, mm[q])
        return acc

    offs0, mm0 = p2fire(0)

    def body(i, carry):
        offs, mm, acc = carry
        offs2, mm2 = p2fire(i + 1)
        wait_gather(i)
        acc = p3(i, offs, mm, acc)
        return offs2, mm2, acc

    offs, mm, acc = lax.fori_loop(
        0, NG - 1, body, (offs0, mm0, jnp.zeros((LN,), jnp.float32)))
    wait_gather(NG - 1)
    acc = p3(NG - 1, offs, mm, acc)
    accv_ref[...] = acc
    pltpu.sync_copy(accv_ref, out_hbm.at[pl.ds(wid * LN, LN)])


@functools.partial(
    pl.kernel,
    out_type=jax.ShapeDtypeStruct((NW * LN,), jnp.float32),
    mesh=plsc.VectorSubcoreMesh(core_axis_name="c", subcore_axis_name="s"),
    compiler_params=pltpu.CompilerParams(
        use_tc_tiling_on_sc=False, needs_layout_passes=False),
    scratch_types=[
        pltpu.VMEM((NCH // 8, 8, 128), jnp.float32),
        pltpu.VMEM((NF, 128), jnp.float32),
        pltpu.VMEM((2 * GL,), jnp.int32),
        pltpu.VMEM((2 * GL, 128), jnp.float32),
        pltpu.VMEM((LN,), jnp.float32),
        pltpu.SemaphoreType.DMA,
    ],
)
def _sc_select(cmt_hbm, smt_hbm, d2l_hbm, out_hbm, cm_ref, sm_ref, idx_ref,
               cand_ref, accv_ref, sem):
    _sc_body(cmt_hbm, smt_hbm, d2l_hbm, out_hbm, cm_ref, sm_ref, idx_ref,
             cand_ref, accv_ref, sem)


def kernel(e, lp):
    del lp
    e_bf = e.astype(jnp.bfloat16)
    d2f, cmt, smt, rs = pl.pallas_call(
        _tc_body,
        grid=(NB,),
        in_specs=[
            pl.BlockSpec((R, D), lambda i: (i, 0)),
            pl.BlockSpec((N, D), lambda i: (0, 0)),
        ],
        out_specs=[
            pl.BlockSpec((R // 8, NF, 8, 128), lambda i: (i, 0, 0, 0)),
            pl.BlockSpec((R // CH // 8, NF, 8, 128), lambda i: (i, 0, 0, 0)),
            pl.BlockSpec((R // 128, NF, 128), lambda i: (i, 0, 0)),
            pl.BlockSpec(memory_space=pltpu.SMEM),
        ],
        out_shape=[
            jax.ShapeDtypeStruct((N // 8, NF, 8, 128), jnp.float32),
            jax.ShapeDtypeStruct((NCH // 8, NF, 8, 128), jnp.float32),
            jax.ShapeDtypeStruct((N // 128, NF, 128), jnp.float32),
            jax.ShapeDtypeStruct((1, 1), jnp.float32),
        ],
        scratch_shapes=[
            pltpu.VMEM((1, N), jnp.float32),
        ],
    )(e_bf, e_bf)
    d2l = d2f.reshape(N // 8 * NF * 8, 128)
    partial = _sc_select(cmt, smt, d2l)
    return (jnp.sum(partial) * (1.0 / (N * K))) / rs[0, 0]


# speedup vs baseline: 2.2719x; 1.0545x over previous
import functools

import jax
import jax.numpy as jnp
from jax import lax
from jax.experimental import pallas as pl
from jax.experimental.pallas import tpu as pltpu
from jax.experimental.pallas import tpu_sc as plsc

N = 4096
D = 128
K = 8
R = 256
NB = N // R
CH = 16
NCH = N // CH
NF = N // 128
LN = 16
NW = 32
RPW = N // NW
NG = RPW // LN
GL = K * LN


def _tc_body(e_all_ref, d2f_ref, cmt_ref, smt_ref, rs_ref, sb_ref, ebf_ref):
    i = pl.program_id(0)

    @pl.when(i == 0)
    def _():
        ebf_ref[...] = e_all_ref[...].astype(jnp.bfloat16)
        ea = ebf_ref[...].astype(jnp.float32)
        sq = ea * ea
        ones = jnp.ones((1, D), dtype=jnp.float32)
        sb_ref[...] = lax.dot_general(
            ones, sq, (((1,), (1,)), ((), ())),
            preferred_element_type=jnp.float32)
        colsum = jnp.sum(ea, axis=0, keepdims=True)
        colsum2 = jnp.sum(sq, axis=0, keepdims=True)
        var = (colsum2 - colsum * colsum * (1.0 / N)) * (1.0 / (N - 1))
        rs_ref[0, 0] = jnp.sum(var) * (1.0 / D)

    e_blk_bf = ebf_ref[pl.ds(i * R, R), :]
    e_blk = e_blk_bf.astype(jnp.float32)
    sa = jnp.sum(e_blk * e_blk, axis=1, keepdims=True)
    g = lax.dot_general(
        e_blk_bf, ebf_ref[...], (((1,), (1,)), ((), ())),
        preferred_element_type=jnp.float32)
    d2 = sa + sb_ref[...] - 2.0 * g
    for k in range(NF):
        d2f_ref[:, k, :, :] = d2[:, k * 128:(k + 1) * 128].reshape(R // 8, 8, 128)
    m = jnp.min(d2.reshape(R // 128, CH, 8, N), axis=1).reshape(R // CH, N)
    for k in range(NF):
        cmt_ref[:, k, :, :] = m[:, k * 128:(k + 1) * 128].reshape(R // CH // 8, 8, 128)
    sm = jnp.min(m.reshape(R // 128, 8, N), axis=1)
    for k in range(NF):
        smt_ref[:, k, :] = sm[:, k * 128:(k + 1) * 128]


def _bcast_i32(x):
    return jnp.zeros((LN,), jnp.int32) + x


def _argmin_vecs(vs, inf16):
    S = 4 if len(vs) >= 8 else 1
    parts = []
    for k in range(S):
        best, bidx = None, None
        for pos in range(k, len(vs), S):
            if best is None:
                best, bidx = vs[pos], _bcast_i32(pos)
            else:
                pred = vs[pos] < best
                best = jnp.where(pred, vs[pos], best)
                bidx = jnp.where(pred, _bcast_i32(pos), bidx)
        parts.append((best, bidx))
    best, bidx = parts[0]
    for b2, i2 in parts[1:]:
        pred = b2 < best
        best = jnp.where(pred, b2, best)
        bidx = jnp.where(pred, i2, bidx)
    return best, bidx


def _min_tree(vs):
    while len(vs) > 1:
        vs = [jnp.minimum(a, b) for a, b in zip(vs[::2], vs[1::2])] + (
            [vs[-1]] if len(vs) % 2 else [])
    return vs[0]


def _sc_body(cmt_hbm, smt_hbm, d2l_hbm, out_hbm, cm_ref, sm_ref, idx_ref,
             cand_ref, accv_ref, sem):
    c = lax.axis_index("c")
    s = lax.axis_index("s")
    wid = s * 2 + c
    lane = lax.broadcasted_iota(jnp.int32, (LN,), 0)
    inf16 = jnp.full((LN,), jnp.inf, jnp.float32)

    pltpu.sync_copy(cmt_hbm.at[:, wid], cm_ref)
    pltpu.sync_copy(smt_hbm.at[:, wid], sm_ref)

    def p2fire(g):
        par = g & 1
        col = g * LN + lane
        row = wid * RPW + g * LN + lane
        offs = []
        mm = []
        for q in range(K):
            svals = [plsc.load_gather(sm_ref, [_bcast_i32(j2), col])
                     for j2 in range(NF)]
            _, sj = _argmin_vecs(svals, inf16)
            vs = [plsc.load_gather(cm_ref, [sj, _bcast_i32(t), col])
                  for t in range(8)]
            cbest, tq = _argmin_vecs(vs, inf16)
            plsc.store_scatter(cm_ref, [sj, tq, col], inf16)
            nm = _min_tree([jnp.where(tq == t, inf16, vs[t])
                            for t in range(8)])
            plsc.store_scatter(sm_ref, [sj, col], nm)
            idx_ref[pl.ds(par * GL + q * LN, LN)] = (
                (row >> 3) * (NF * 8) + sj * 8 + (row & 7))
            offs.append(tq)
            mm.append(cbest)
        pltpu.async_copy(d2l_hbm.at[idx_ref.at[pl.ds(par * GL, GL)]],
                         cand_ref.at[pl.ds(par * GL, GL)], sem)
        return tuple(offs), tuple(mm)

    def wait_gather(g):
        par = g & 1
        pltpu.make_async_copy(d2l_hbm.at[idx_ref.at[pl.ds(par * GL, GL)]],
                              cand_ref.at[pl.ds(par * GL, GL)], sem).wait()

    def p3(g, offs, mm, acc):
        par = g & 1
        mm = list(mm)
        for p in range(K):
            _, qb = _argmin_vecs(mm, inf16)
            rowi = _bcast_i32(par * GL) + qb * LN + lane
            offsel = offs[0]
            for q in range(1, K):
                offsel = jnp.where(qb == q, offs[q], offsel)
            vs = [plsc.load_gather(cand_ref, [rowi, offsel + t * 8])
                  for t in range(CH)]
            vbest, ti = _argmin_vecs(vs, inf16)
            acc = acc + vbest
            plsc.store_scatter(cand_ref, [rowi, offsel + ti * 8], inf16)
            nm = _min_tree([jnp.where(ti == t, inf16, vs[t])
                            for t in range(CH)])
            for q in range(K):
                mm[q] = jnp.where(qb == q, nm, mm[q])
        return acc

    offs0, mm0 = p2fire(0)

    def body(i, carry):
        offs, mm, acc = carry
        offs2, mm2 = p2fire(i + 1)
        wait_gather(i)
        acc = p3(i, offs, mm, acc)
        return offs2, mm2, acc

    offs, mm, acc = lax.fori_loop(
        0, NG - 1, body, (offs0, mm0, jnp.zeros((LN,), jnp.float32)))
    wait_gather(NG - 1)
    acc = p3(NG - 1, offs, mm, acc)
    accv_ref[...] = acc
    pltpu.sync_copy(accv_ref, out_hbm.at[pl.ds(wid * LN, LN)])


@functools.partial(
    pl.kernel,
    out_type=jax.ShapeDtypeStruct((NW * LN,), jnp.float32),
    mesh=plsc.VectorSubcoreMesh(core_axis_name="c", subcore_axis_name="s"),
    compiler_params=pltpu.CompilerParams(
        use_tc_tiling_on_sc=False, needs_layout_passes=False),
    scratch_types=[
        pltpu.VMEM((NCH // 8, 8, 128), jnp.float32),
        pltpu.VMEM((NF, 128), jnp.float32),
        pltpu.VMEM((2 * GL,), jnp.int32),
        pltpu.VMEM((2 * GL, 128), jnp.float32),
        pltpu.VMEM((LN,), jnp.float32),
        pltpu.SemaphoreType.DMA,
    ],
)
def _sc_select(cmt_hbm, smt_hbm, d2l_hbm, out_hbm, cm_ref, sm_ref, idx_ref,
               cand_ref, accv_ref, sem):
    _sc_body(cmt_hbm, smt_hbm, d2l_hbm, out_hbm, cm_ref, sm_ref, idx_ref,
             cand_ref, accv_ref, sem)


def kernel(e, lp):
    del lp
    d2f, cmt, smt, rs = pl.pallas_call(
        _tc_body,
        grid=(NB,),
        in_specs=[
            pl.BlockSpec((N, D), lambda i: (0, 0)),
        ],
        out_specs=[
            pl.BlockSpec((R // 8, NF, 8, 128), lambda i: (i, 0, 0, 0)),
            pl.BlockSpec((R // CH // 8, NF, 8, 128), lambda i: (i, 0, 0, 0)),
            pl.BlockSpec((R // 128, NF, 128), lambda i: (i, 0, 0)),
            pl.BlockSpec(memory_space=pltpu.SMEM),
        ],
        out_shape=[
            jax.ShapeDtypeStruct((N // 8, NF, 8, 128), jnp.float32),
            jax.ShapeDtypeStruct((NCH // 8, NF, 8, 128), jnp.float32),
            jax.ShapeDtypeStruct((N // 128, NF, 128), jnp.float32),
            jax.ShapeDtypeStruct((1, 1), jnp.float32),
        ],
        scratch_shapes=[
            pltpu.VMEM((1, N), jnp.float32),
            pltpu.VMEM((N, D), jnp.bfloat16),
        ],
    )(e)
    d2l = d2f.reshape(N // 8 * NF * 8, 128)
    partial = _sc_select(cmt, smt, d2l)
    return (jnp.sum(partial) * (1.0 / (N * K))) / rs[0, 0]


# speedup vs baseline: 2.4124x; 1.0618x over previous
import functools

import jax
import jax.numpy as jnp
from jax import lax
from jax.experimental import pallas as pl
from jax.experimental.pallas import tpu as pltpu
from jax.experimental.pallas import tpu_sc as plsc

N = 4096
D = 128
K = 8
R = 256
NB = N // R
CH = 16
NCH = N // CH
NF = N // 128
LN = 16
NW = 32
RPW = N // NW
NG = RPW // LN
GL = K * LN


def _tc_body(e_all_ref, d2f_ref, cmt_ref, smt_ref, rs_ref, sb_ref, ebf_ref):
    i = pl.program_id(0)

    @pl.when(i == 0)
    def _():
        ebf_ref[...] = e_all_ref[...].astype(jnp.bfloat16)
        ea = ebf_ref[...].astype(jnp.float32)
        sq = ea * ea
        ones = jnp.ones((1, D), dtype=jnp.float32)
        sb_ref[...] = lax.dot_general(
            ones, sq, (((1,), (1,)), ((), ())),
            preferred_element_type=jnp.float32)
        colsum = jnp.sum(ea, axis=0, keepdims=True)
        colsum2 = jnp.sum(sq, axis=0, keepdims=True)
        var = (colsum2 - colsum * colsum * (1.0 / N)) * (1.0 / (N - 1))
        rs_ref[0, 0] = jnp.sum(var) * (1.0 / D)

    e_blk_bf = ebf_ref[pl.ds(i * R, R), :]
    e_blk = e_blk_bf.astype(jnp.float32)
    sa = jnp.sum(e_blk * e_blk, axis=1, keepdims=True)
    g = lax.dot_general(
        e_blk_bf, ebf_ref[...], (((1,), (1,)), ((), ())),
        preferred_element_type=jnp.float32)
    d2 = sa + sb_ref[...] - 2.0 * g
    w32 = pltpu.bitcast(d2.astype(jnp.bfloat16), jnp.int32)
    for k in range(NF):
        d2f_ref[:, k, :, :] = w32[:, k * 128:(k + 1) * 128].reshape(R // 16, 8, 128)
    m = jnp.min(d2.reshape(R // 128, CH, 8, N), axis=1).reshape(R // CH, N)
    for k in range(NF):
        cmt_ref[:, k, :, :] = m[:, k * 128:(k + 1) * 128].reshape(R // CH // 8, 8, 128)
    sm = jnp.min(m.reshape(R // 128, 8, N), axis=1)
    for k in range(NF):
        smt_ref[:, k, :] = sm[:, k * 128:(k + 1) * 128]


def _bcast_i32(x):
    return jnp.zeros((LN,), jnp.int32) + x


def _argmin_vecs(vs, inf16):
    S = 4 if len(vs) >= 8 else 1
    parts = []
    for k in range(S):
        best, bidx = None, None
        for pos in range(k, len(vs), S):
            if best is None:
                best, bidx = vs[pos], _bcast_i32(pos)
            else:
                pred = vs[pos] < best
                best = jnp.where(pred, vs[pos], best)
                bidx = jnp.where(pred, _bcast_i32(pos), bidx)
        parts.append((best, bidx))
    best, bidx = parts[0]
    for b2, i2 in parts[1:]:
        pred = b2 < best
        best = jnp.where(pred, b2, best)
        bidx = jnp.where(pred, i2, bidx)
    return best, bidx


def _min_tree(vs):
    while len(vs) > 1:
        vs = [jnp.minimum(a, b) for a, b in zip(vs[::2], vs[1::2])] + (
            [vs[-1]] if len(vs) % 2 else [])
    return vs[0]


def _sc_body(cmt_hbm, smt_hbm, d2l_hbm, out_hbm, cm_ref, sm_ref, idx_ref,
             cand_ref, accv_ref, sem):
    c = lax.axis_index("c")
    s = lax.axis_index("s")
    wid = s * 2 + c
    lane = lax.broadcasted_iota(jnp.int32, (LN,), 0)
    inf16 = jnp.full((LN,), jnp.inf, jnp.float32)

    pltpu.sync_copy(cmt_hbm.at[:, wid], cm_ref)
    pltpu.sync_copy(smt_hbm.at[:, wid], sm_ref)

    def p2fire(g):
        par = g & 1
        col = g * LN + lane
        row = wid * RPW + g * LN + lane
        offs = []
        mm = []
        for q in range(K):
            svals = [plsc.load_gather(sm_ref, [_bcast_i32(j2), col])
                     for j2 in range(NF)]
            _, sj = _argmin_vecs(svals, inf16)
            vs = [plsc.load_gather(cm_ref, [sj, _bcast_i32(t), col])
                  for t in range(8)]
            cbest, tq = _argmin_vecs(vs, inf16)
            plsc.store_scatter(cm_ref, [sj, tq, col], inf16)
            nm = _min_tree([jnp.where(tq == t, inf16, vs[t])
                            for t in range(8)])
            plsc.store_scatter(sm_ref, [sj, col], nm)
            idx_ref[pl.ds(par * GL + q * LN, LN)] = (
                (row >> 4) * (NF * 8) + sj * 8 + ((row >> 1) & 7))
            offs.append(tq)
            mm.append(cbest)
        pltpu.async_copy(d2l_hbm.at[idx_ref.at[pl.ds(par * GL, GL)]],
                         cand_ref.at[pl.ds(par * GL, GL)], sem)
        return tuple(offs), tuple(mm)

    def wait_gather(g):
        par = g & 1
        pltpu.make_async_copy(d2l_hbm.at[idx_ref.at[pl.ds(par * GL, GL)]],
                              cand_ref.at[pl.ds(par * GL, GL)], sem).wait()

    lane_even = (lane & 1) == 0
    inf_word = jnp.full((LN,), 0x7f807f80, jnp.int32)
    himask = jnp.full((LN,), -65536, jnp.int32)

    def decode(w):
        return plsc.bitcast(
            jnp.where(lane_even, w << 16, w & himask), jnp.float32)

    def p3(g, offs, mm, acc):
        par = g & 1
        mm = list(mm)
        for p in range(K):
            _, qb = _argmin_vecs(mm, inf16)
            rowi = _bcast_i32(par * GL) + qb * LN + lane
            offsel = offs[0]
            for q in range(1, K):
                offsel = jnp.where(qb == q, offs[q], offsel)
            vs = [decode(plsc.load_gather(cand_ref, [rowi, offsel + t * 8]))
                  for t in range(CH)]
            vbest, ti = _argmin_vecs(vs, inf16)
            acc = acc + vbest
            plsc.store_scatter(cand_ref, [rowi, offsel + ti * 8], inf_word)
            nm = _min_tree([jnp.where(ti == t, inf16, vs[t])
                            for t in range(CH)])
            for q in range(K):
                mm[q] = jnp.where(qb == q, nm, mm[q])
        return acc

    offs0, mm0 = p2fire(0)

    def body(i, carry):
        offs, mm, acc = carry
        offs2, mm2 = p2fire(i + 1)
        wait_gather(i)
        acc = p3(i, offs, mm, acc)
        return offs2, mm2, acc

    offs, mm, acc = lax.fori_loop(
        0, NG - 1, body, (offs0, mm0, jnp.zeros((LN,), jnp.float32)))
    wait_gather(NG - 1)
    acc = p3(NG - 1, offs, mm, acc)
    accv_ref[...] = acc
    pltpu.sync_copy(accv_ref, out_hbm.at[pl.ds(wid * LN, LN)])


@functools.partial(
    pl.kernel,
    out_type=jax.ShapeDtypeStruct((NW * LN,), jnp.float32),
    mesh=plsc.VectorSubcoreMesh(core_axis_name="c", subcore_axis_name="s"),
    compiler_params=pltpu.CompilerParams(
        use_tc_tiling_on_sc=False, needs_layout_passes=False),
    scratch_types=[
        pltpu.VMEM((NCH // 8, 8, 128), jnp.float32),
        pltpu.VMEM((NF, 128), jnp.float32),
        pltpu.VMEM((2 * GL,), jnp.int32),
        pltpu.VMEM((2 * GL, 128), jnp.int32),
        pltpu.VMEM((LN,), jnp.float32),
        pltpu.SemaphoreType.DMA,
    ],
)
def _sc_select(cmt_hbm, smt_hbm, d2l_hbm, out_hbm, cm_ref, sm_ref, idx_ref,
               cand_ref, accv_ref, sem):
    _sc_body(cmt_hbm, smt_hbm, d2l_hbm, out_hbm, cm_ref, sm_ref, idx_ref,
             cand_ref, accv_ref, sem)


def kernel(e, lp):
    del lp
    d2f, cmt, smt, rs = pl.pallas_call(
        _tc_body,
        grid=(NB,),
        in_specs=[
            pl.BlockSpec((N, D), lambda i: (0, 0)),
        ],
        out_specs=[
            pl.BlockSpec((R // 16, NF, 8, 128), lambda i: (i, 0, 0, 0)),
            pl.BlockSpec((R // CH // 8, NF, 8, 128), lambda i: (i, 0, 0, 0)),
            pl.BlockSpec((R // 128, NF, 128), lambda i: (i, 0, 0)),
            pl.BlockSpec(memory_space=pltpu.SMEM),
        ],
        out_shape=[
            jax.ShapeDtypeStruct((N // 16, NF, 8, 128), jnp.int32),
            jax.ShapeDtypeStruct((NCH // 8, NF, 8, 128), jnp.float32),
            jax.ShapeDtypeStruct((N // 128, NF, 128), jnp.float32),
            jax.ShapeDtypeStruct((1, 1), jnp.float32),
        ],
        scratch_shapes=[
            pltpu.VMEM((1, N), jnp.float32),
            pltpu.VMEM((N, D), jnp.bfloat16),
        ],
    )(e)
    d2l = d2f.reshape(N // 16 * NF * 8, 128)
    partial = _sc_select(cmt, smt, d2l)
    return (jnp.sum(partial) * (1.0 / (N * K))) / rs[0, 0]
